# baseline (speedup 1.0000x reference)
import jax
import jax.numpy as jnp
from jax.experimental import pallas as pl
from jax.experimental.pallas import tpu as pltpu

_NUM_GROUPS = 32
_EPS = 1e-5
_BB = 1024
_HL = 256
_CC = 128


def _gemm_half(xh, w_ref):
    c, k = w_ref.shape
    half = c // 2
    n_kt = k // 256
    for kt in range(n_kt):
        sr = kt % 2
        rhs = xh[:, kt * 256:(kt + 1) * 256]
        pltpu.matmul_push_rhs(rhs, staging_register=sr, mxu_index=0,
                              transpose=True)
        pltpu.matmul_push_rhs(rhs, staging_register=sr, mxu_index=1,
                              transpose=True)
        pltpu.matmul_acc_lhs(0, w_ref[0:half, kt * 256:(kt + 1) * 256],
                             mxu_index=0, load_staged_rhs=sr)
        pltpu.matmul_acc_lhs(0, w_ref[half:c, kt * 256:(kt + 1) * 256],
                             mxu_index=1, load_staged_rhs=sr)
    y_top = pltpu.matmul_pop(0, (half, _HL), jnp.float32, mxu_index=0)
    y_bot = pltpu.matmul_pop(0, (half, _HL), jnp.float32, mxu_index=1)
    return jnp.concatenate([y_top, y_bot], axis=0)


def _gn_min(y, lb, gw, gb):
    c, bb = y.shape
    g = _NUM_GROUPS
    gs = c // g
    y = y + lb
    yg = y.reshape(g, gs, bb)
    mean = jnp.mean(yg, axis=1, keepdims=True)
    msq = jnp.mean(yg * yg, axis=1, keepdims=True)
    var = msq - mean * mean
    r = jax.lax.rsqrt(var + _EPS)
    yhat = ((yg - mean) * r).reshape(c, bb)
    ynorm = yhat * gw + gb
    return jnp.min(ynorm, axis=0, keepdims=True)


def _minred_kernel(x_ref, w_ref, lb_ref, gw_ref, gb_ref, m_ref):
    lb = lb_ref[...]
    gw = gw_ref[...]
    gb = gb_ref[...]
    xb = x_ref[...].astype(jnp.bfloat16)
    for h in range(_BB // _HL):
        xh = xb[h * _HL:(h + 1) * _HL, :]
        y = _gemm_half(xh, w_ref)
        mins = _gn_min(y, lb, gw, gb)
        m_ref[:, h * _HL:(h + 1) * _HL] = jnp.broadcast_to(mins, (8, _HL))


def _bcast_kernel(m_ref, fb_ref, o_ref):
    cc = fb_ref.shape[0]
    bt, lanes = m_ref.shape
    mins3 = jnp.broadcast_to(m_ref[...][None, :, :], (cc, bt, lanes))
    bias3 = jnp.broadcast_to(fb_ref[...][:, None, :], (cc, bt, lanes))
    o_ref[...] = (mins3 + bias3).reshape(cc * bt, lanes)


def kernel(x, weight, linear_bias, gn_weight, gn_bias, final_bias):
    b, k = x.shape
    c = weight.shape[0]
    bb = _BB
    cc = _CC

    wb = weight.astype(jnp.bfloat16)
    lb = jnp.broadcast_to(linear_bias[:, None], (c, _HL))
    gw = jnp.broadcast_to(gn_weight[:, None], (c, _HL))
    gb = jnp.broadcast_to(gn_bias[:, None], (c, _HL))
    fb = jnp.broadcast_to(final_bias.reshape(c)[:, None], (c, 128))

    nb = b // bb
    mins8 = pl.pallas_call(
        _minred_kernel,
        grid=(nb,),
        in_specs=[
            pl.BlockSpec((bb, k), lambda i: (i, 0)),
            pl.BlockSpec((c, k), lambda i: (0, 0)),
            pl.BlockSpec((c, _HL), lambda i: (0, 0)),
            pl.BlockSpec((c, _HL), lambda i: (0, 0)),
            pl.BlockSpec((c, _HL), lambda i: (0, 0)),
        ],
        out_specs=pl.BlockSpec((8, bb), lambda i: (0, i)),
        out_shape=jax.ShapeDtypeStruct((8, b), jnp.float32),
        compiler_params=pltpu.CompilerParams(
            dimension_semantics=("parallel",),
            vmem_limit_bytes=56 * 1024 * 1024,
        ),
    )(x, wb, lb, gw, gb)

    mins2d = mins8[0].reshape(b // 128, 128)

    nc = c // cc
    out_lin = pl.pallas_call(
        _bcast_kernel,
        grid=(nc,),
        in_specs=[
            pl.BlockSpec((b // 128, 128), lambda j: (0, 0)),
            pl.BlockSpec((cc, 128), lambda j: (j, 0)),
        ],
        out_specs=pl.BlockSpec((cc * (b // 128), 128), lambda j: (j, 0)),
        out_shape=jax.ShapeDtypeStruct((c * (b // 128), 128), jnp.float32),
        compiler_params=pltpu.CompilerParams(
            dimension_semantics=("parallel",),
            vmem_limit_bytes=56 * 1024 * 1024,
        ),
    )(mins2d, fb)

    return out_lin.reshape(1, c, b, 1)


# speedup vs baseline: 1.0223x; 1.0223x over previous
import jax
import jax.numpy as jnp
from jax.experimental import pallas as pl
from jax.experimental.pallas import tpu as pltpu

_NUM_GROUPS = 32
_EPS = 1e-5
_BB = 2048
_HL = 256
_CC = 128


def _gemm_half(xh, w_ref):
    c, k = w_ref.shape
    half = c // 2
    n_kt = k // 256
    for kt in range(n_kt):
        sr = kt % 2
        rhs = xh[:, kt * 256:(kt + 1) * 256]
        pltpu.matmul_push_rhs(rhs, staging_register=sr, mxu_index=0,
                              transpose=True)
        pltpu.matmul_push_rhs(rhs, staging_register=sr, mxu_index=1,
                              transpose=True)
        pltpu.matmul_acc_lhs(0, w_ref[0:half, kt * 256:(kt + 1) * 256],
                             mxu_index=0, load_staged_rhs=sr)
        pltpu.matmul_acc_lhs(0, w_ref[half:c, kt * 256:(kt + 1) * 256],
                             mxu_index=1, load_staged_rhs=sr)
    y_top = pltpu.matmul_pop(0, (half, _HL), jnp.float32, mxu_index=0)
    y_bot = pltpu.matmul_pop(0, (half, _HL), jnp.float32, mxu_index=1)
    return jnp.concatenate([y_top, y_bot], axis=0)


def _gn_min(y, lb, gw, gb):
    c, bb = y.shape
    g = _NUM_GROUPS
    gs = c // g
    y = y + lb
    yg = y.reshape(g, gs, bb)
    mean = jnp.mean(yg, axis=1, keepdims=True)
    msq = jnp.mean(yg * yg, axis=1, keepdims=True)
    var = msq - mean * mean
    r = jax.lax.rsqrt(var + _EPS)
    yhat = ((yg - mean) * r).reshape(c, bb)
    ynorm = yhat * gw + gb
    return jnp.min(ynorm, axis=0, keepdims=True)


def _minred_kernel(x_ref, w_ref, lb_ref, gw_ref, gb_ref, m_ref):
    lb = lb_ref[...]
    gw = gw_ref[...]
    gb = gb_ref[...]
    xb = x_ref[...].astype(jnp.bfloat16)
    for h in range(_BB // _HL):
        xh = xb[h * _HL:(h + 1) * _HL, :]
        y = _gemm_half(xh, w_ref)
        mins = _gn_min(y, lb, gw, gb)
        m_ref[:, h * _HL:(h + 1) * _HL] = jnp.broadcast_to(mins, (8, _HL))


def _bcast_kernel(m_ref, fb_ref, o_ref):
    cc = fb_ref.shape[0]
    bt, lanes = m_ref.shape
    mins3 = jnp.broadcast_to(m_ref[...][None, :, :], (cc, bt, lanes))
    bias3 = jnp.broadcast_to(fb_ref[...][:, None, :], (cc, bt, lanes))
    o_ref[...] = (mins3 + bias3).reshape(cc * bt, lanes)


def kernel(x, weight, linear_bias, gn_weight, gn_bias, final_bias):
    b, k = x.shape
    c = weight.shape[0]
    bb = _BB
    cc = _CC

    wb = weight.astype(jnp.bfloat16)
    lb = jnp.broadcast_to(linear_bias[:, None], (c, _HL))
    gw = jnp.broadcast_to(gn_weight[:, None], (c, _HL))
    gb = jnp.broadcast_to(gn_bias[:, None], (c, _HL))
    fb = jnp.broadcast_to(final_bias.reshape(c)[:, None], (c, 128))

    nb = b // bb
    mins8 = pl.pallas_call(
        _minred_kernel,
        grid=(nb,),
        in_specs=[
            pl.BlockSpec((bb, k), lambda i: (i, 0)),
            pl.BlockSpec((c, k), lambda i: (0, 0)),
            pl.BlockSpec((c, _HL), lambda i: (0, 0)),
            pl.BlockSpec((c, _HL), lambda i: (0, 0)),
            pl.BlockSpec((c, _HL), lambda i: (0, 0)),
        ],
        out_specs=pl.BlockSpec((8, bb), lambda i: (0, i)),
        out_shape=jax.ShapeDtypeStruct((8, b), jnp.float32),
        compiler_params=pltpu.CompilerParams(
            dimension_semantics=("parallel",),
            vmem_limit_bytes=56 * 1024 * 1024,
        ),
    )(x, wb, lb, gw, gb)

    mins2d = mins8[0].reshape(b // 128, 128)

    nc = c // cc
    out_lin = pl.pallas_call(
        _bcast_kernel,
        grid=(nc,),
        in_specs=[
            pl.BlockSpec((b // 128, 128), lambda j: (0, 0)),
            pl.BlockSpec((cc, 128), lambda j: (j, 0)),
        ],
        out_specs=pl.BlockSpec((cc * (b // 128), 128), lambda j: (j, 0)),
        out_shape=jax.ShapeDtypeStruct((c * (b // 128), 128), jnp.float32),
        compiler_params=pltpu.CompilerParams(
            dimension_semantics=("parallel",),
            vmem_limit_bytes=56 * 1024 * 1024,
        ),
    )(mins2d, fb)

    return out_lin.reshape(1, c, b, 1)
